# Initial kernel scaffold; baseline (speedup 1.0000x reference)
import jax
import jax.numpy as jnp
from jax.experimental import pallas as pl


def kernel(x, edge_index, W1, b1, W2, b2):
    raise NotImplementedError("write your pallas kernel here")


# speedup vs baseline: 20.7437x; 20.7437x over previous
import functools

import jax
import jax.numpy as jnp
from jax import lax
from jax.experimental import pallas as pl
from jax.experimental.pallas import tpu as pltpu
from jax.experimental.pallas import tpu_sc as plsc

N = 10000
E = 320000
IN_CH = 128
HID = 128
OUT_CH = 64

NC = 2
NS = 16
NW = NC * NS
EPW = E // NW
CH = 80
NCHUNK = EPW // CH
NPAD = 10240
RPT = NPAD // NS

_MESH = plsc.VectorSubcoreMesh(core_axis_name="c", subcore_axis_name="s")
_SC_PARAMS = pltpu.CompilerParams(use_tc_tiling_on_sc=False)


def _deg_body(col_hbm, ones_hbm, zeros_hbm, out_hbm, col_v, ones_v, hist_s):
    cid = lax.axis_index("c")
    sid = lax.axis_index("s")
    wid = cid * NS + sid
    pltpu.sync_copy(zeros_hbm.at[pl.ds(sid * RPT, RPT)],
                    hist_s.at[pl.ds(sid * RPT, RPT)])
    pltpu.sync_copy(col_hbm.at[wid], col_v)
    pltpu.sync_copy(ones_hbm, ones_v)
    plsc.subcore_barrier()

    @pl.loop(0, NCHUNK)
    def _(j):
        pltpu.sync_copy(ones_v, hist_s.at[col_v.at[j]], add=True)

    plsc.subcore_barrier()
    pltpu.sync_copy(hist_s.at[pl.ds(sid * RPT, RPT)],
                    out_hbm.at[cid, pl.ds(sid * RPT, RPT)])


def _prop_body(g_hbm, row_hbm, col_hbm, zeros_hbm, out_hbm,
               row_v, col_v, rows_v, acc_s):
    cid = lax.axis_index("c")
    sid = lax.axis_index("s")
    wid = cid * NS + sid
    pltpu.sync_copy(zeros_hbm.at[pl.ds(sid * RPT, RPT)],
                    acc_s.at[pl.ds(sid * RPT, RPT)])
    pltpu.sync_copy(row_hbm.at[wid], row_v)
    pltpu.sync_copy(col_hbm.at[wid], col_v)
    plsc.subcore_barrier()

    @pl.loop(0, NCHUNK)
    def _(j):
        pltpu.sync_copy(g_hbm.at[row_v.at[j]], rows_v)
        pltpu.sync_copy(rows_v, acc_s.at[col_v.at[j]], add=True)

    plsc.subcore_barrier()
    pltpu.sync_copy(acc_s.at[pl.ds(sid * RPT, RPT)],
                    out_hbm.at[cid, pl.ds(sid * RPT, RPT)])


def _deg_call(col, ones, zeros):
    k = pl.kernel(
        _deg_body,
        out_type=jax.ShapeDtypeStruct((NC, NPAD, 16), jnp.float32),
        mesh=_MESH,
        scratch_types=[
            pltpu.VMEM((NCHUNK, CH), jnp.int32),
            pltpu.VMEM((CH, 16), jnp.float32),
            pltpu.VMEM_SHARED((NPAD, 16), jnp.float32),
        ],
        compiler_params=_SC_PARAMS,
    )
    return k(col, ones, zeros)


def _prop_call(g, row, col, zeros, f):
    k = pl.kernel(
        _prop_body,
        out_type=jax.ShapeDtypeStruct((NC, NPAD, f), jnp.float32),
        mesh=_MESH,
        scratch_types=[
            pltpu.VMEM((NCHUNK, CH), jnp.int32),
            pltpu.VMEM((NCHUNK, CH), jnp.int32),
            pltpu.VMEM((CH, f), jnp.float32),
            pltpu.VMEM_SHARED((NPAD, f), jnp.float32),
        ],
        compiler_params=_SC_PARAMS,
    )
    return k(g, row, col, zeros)


_BLK = 1000


def _dinv_from(degp_ref):
    deg = 1.0 + degp_ref[0, :, 0:1] + degp_ref[1, :, 0:1]
    return lax.rsqrt(deg)


def _tc_a_body(x_ref, w_ref, degp_ref, g_ref):
    h = jnp.dot(x_ref[...], w_ref[...], preferred_element_type=jnp.float32,
                precision=lax.Precision.HIGHEST)
    g_ref[...] = h * _dinv_from(degp_ref)


def _tc_a_call(x, w1, degp):
    return pl.pallas_call(
        _tc_a_body,
        grid=(N // _BLK,),
        in_specs=[
            pl.BlockSpec((_BLK, IN_CH), lambda i: (i, 0)),
            pl.BlockSpec((IN_CH, HID), lambda i: (0, 0)),
            pl.BlockSpec((NC, _BLK, 16), lambda i: (0, i, 0)),
        ],
        out_specs=pl.BlockSpec((_BLK, HID), lambda i: (i, 0)),
        out_shape=jax.ShapeDtypeStruct((N, HID), jnp.float32),
    )(x, w1, degp)


def _tc_b_body(p_ref, g1_ref, degp_ref, b1_ref, w2_ref, g2_ref):
    dinv = _dinv_from(degp_ref)
    s = dinv * (p_ref[0] + p_ref[1] + g1_ref[...]) + b1_ref[...]
    t = jnp.tanh(s)
    h2 = jnp.dot(t, w2_ref[...], preferred_element_type=jnp.float32,
                 precision=lax.Precision.HIGHEST)
    g2_ref[...] = h2 * dinv


def _tc_b_call(p1, g1, degp, b1, w2):
    return pl.pallas_call(
        _tc_b_body,
        grid=(N // _BLK,),
        in_specs=[
            pl.BlockSpec((NC, _BLK, HID), lambda i: (0, i, 0)),
            pl.BlockSpec((_BLK, HID), lambda i: (i, 0)),
            pl.BlockSpec((NC, _BLK, 16), lambda i: (0, i, 0)),
            pl.BlockSpec((1, HID), lambda i: (0, 0)),
            pl.BlockSpec((HID, OUT_CH), lambda i: (0, 0)),
        ],
        out_specs=pl.BlockSpec((_BLK, OUT_CH), lambda i: (i, 0)),
        out_shape=jax.ShapeDtypeStruct((N, OUT_CH), jnp.float32),
    )(p1, g1, degp, b1, w2)


def _tc_c_body(q_ref, g2_ref, degp_ref, b2_ref, o_ref):
    dinv = _dinv_from(degp_ref)
    o_ref[...] = dinv * (q_ref[0] + q_ref[1] + g2_ref[...]) + b2_ref[...]


def _tc_c_call(p2, g2, degp, b2):
    return pl.pallas_call(
        _tc_c_body,
        grid=(N // _BLK,),
        in_specs=[
            pl.BlockSpec((NC, _BLK, OUT_CH), lambda i: (0, i, 0)),
            pl.BlockSpec((_BLK, OUT_CH), lambda i: (i, 0)),
            pl.BlockSpec((NC, _BLK, 16), lambda i: (0, i, 0)),
            pl.BlockSpec((1, OUT_CH), lambda i: (0, 0)),
        ],
        out_specs=pl.BlockSpec((_BLK, OUT_CH), lambda i: (i, 0)),
        out_shape=jax.ShapeDtypeStruct((N, OUT_CH), jnp.float32),
    )(p2, g2, degp, b2)


def kernel(x, edge_index, W1, b1, W2, b2):
    row = edge_index[0].reshape(NW, NCHUNK, CH)
    col = edge_index[1].reshape(NW, NCHUNK, CH)
    ones16 = jnp.ones((CH, 16), jnp.float32)
    zeros16 = jnp.zeros((NPAD, 16), jnp.float32)
    zeros_hid = jnp.zeros((NPAD, HID), jnp.float32)
    zeros_out = jnp.zeros((NPAD, OUT_CH), jnp.float32)

    degp = _deg_call(col, ones16, zeros16)
    g1 = _tc_a_call(x, W1, degp)
    p1 = _prop_call(g1, row, col, zeros_hid, HID)
    g2 = _tc_b_call(p1, g1, degp, b1.reshape(1, HID), W2)
    p2 = _prop_call(g2, row, col, zeros_out, OUT_CH)
    return _tc_c_call(p2, g2, degp, b2.reshape(1, OUT_CH))
